# Initial kernel scaffold; baseline (speedup 1.0000x reference)
import jax
import jax.numpy as jnp
from jax import lax
from jax.experimental import pallas as pl
from jax.experimental.pallas import tpu as pltpu
from jax.experimental.pallas import tpu_sc as plsc

_NC, _NS = 2, 16
_NW = _NC * _NS
_CH = 128


def _emb_body(x_hbm, w_hbm, out_hbm, idx_v, rows_v, gsem):
    wid = lax.axis_index("s") * _NC + lax.axis_index("c")
    k = idx_v.shape[0]
    pltpu.sync_copy(x_hbm.at[pl.ds(wid * k, k)], idx_v)

    def step(j, carry):
        pltpu.async_copy(w_hbm.at[idx_v.at[j]], rows_v, gsem).wait()
        pltpu.sync_copy(rows_v, out_hbm.at[pl.ds((wid * k + j) * _CH, _CH)])
        return carry

    lax.fori_loop(0, k, step, 0)


def kernel(x, W):
    b, s = x.shape
    v, d = W.shape
    n = b * s
    k = n // (_NW * _CH)
    x2 = x.reshape(_NW * k, _CH)
    mesh = plsc.VectorSubcoreMesh(core_axis_name="c", subcore_axis_name="s")
    out = pl.kernel(
        _emb_body,
        out_type=jax.ShapeDtypeStruct((n, d), jnp.float32),
        mesh=mesh,
        scratch_types=[
            pltpu.VMEM((k, _CH), jnp.int32),
            pltpu.VMEM((_CH, d), jnp.float32),
            pltpu.SemaphoreType.DMA,
        ],
    )(x2, W)
    return out.reshape(b, s, d)


# speedup vs baseline: 1.3063x; 1.3063x over previous
import jax
import jax.numpy as jnp
from jax import lax
from jax.experimental import pallas as pl
from jax.experimental.pallas import tpu as pltpu
from jax.experimental.pallas import tpu_sc as plsc

_NC, _NS = 2, 16
_NW = _NC * _NS
_CH = 128


def _emb_body(x_hbm, w_hbm, out_hbm, idx_v, rows_v, gsem):
    wid = lax.axis_index("s") * _NC + lax.axis_index("c")
    k = idx_v.shape[0]
    pltpu.sync_copy(x_hbm.at[pl.ds(wid * k, k)], idx_v)

    def step(j, carry):
        pltpu.async_copy(w_hbm.at[idx_v.at[j]], rows_v, gsem).wait()
        pltpu.sync_copy(rows_v, out_hbm.at[pl.ds((wid * k + j) * _CH, _CH)])
        return carry

    lax.fori_loop(0, k, step, 0)


def kernel(x, W):
    b, s = x.shape
    v, d = W.shape
    n = b * s
    k = n // (_NW * _CH)
    x2 = x.reshape(_NW * k, _CH)
    mesh = plsc.VectorSubcoreMesh(core_axis_name="c", subcore_axis_name="s")
    out = pl.kernel(
        _emb_body,
        out_type=jax.ShapeDtypeStruct((n, d), jnp.float32),
        mesh=mesh,
        scratch_types=[
            pltpu.VMEM((k, _CH), jnp.int32),
            pltpu.VMEM((_CH, d), jnp.float32),
            pltpu.SemaphoreType.DMA,
        ],
        compiler_params=pltpu.CompilerParams(use_tc_tiling_on_sc=False),
    )(x2, W)
    return out.reshape(b, s, d)


# speedup vs baseline: 1.4767x; 1.1304x over previous
import jax
import jax.numpy as jnp
from jax import lax
from jax.experimental import pallas as pl
from jax.experimental.pallas import tpu as pltpu
from jax.experimental.pallas import tpu_sc as plsc

_NC, _NS = 2, 16
_NW = _NC * _NS
_CH = 1024


def _emb_body(x_hbm, w_hbm, out_hbm, idx_v, rows_v, gsem):
    wid = lax.axis_index("s") * _NC + lax.axis_index("c")
    k = idx_v.shape[0]
    pltpu.sync_copy(x_hbm.at[pl.ds(wid * k, k)], idx_v)

    def step(j, carry):
        pltpu.async_copy(w_hbm.at[idx_v.at[j]], rows_v, gsem).wait()
        pltpu.sync_copy(rows_v, out_hbm.at[pl.ds((wid * k + j) * _CH, _CH)])
        return carry

    lax.fori_loop(0, k, step, 0)


def kernel(x, W):
    b, s = x.shape
    v, d = W.shape
    n = b * s
    k = n // (_NW * _CH)
    x2 = x.reshape(_NW * k, _CH)
    mesh = plsc.VectorSubcoreMesh(core_axis_name="c", subcore_axis_name="s")
    out = pl.kernel(
        _emb_body,
        out_type=jax.ShapeDtypeStruct((n, d), jnp.float32),
        mesh=mesh,
        scratch_types=[
            pltpu.VMEM((k, _CH), jnp.int32),
            pltpu.VMEM((_CH, d), jnp.float32),
            pltpu.SemaphoreType.DMA,
        ],
        compiler_params=pltpu.CompilerParams(use_tc_tiling_on_sc=False),
    )(x2, W)
    return out.reshape(b, s, d)


# speedup vs baseline: 1.5022x; 1.0172x over previous
import jax
import jax.numpy as jnp
from jax import lax
from jax.experimental import pallas as pl
from jax.experimental.pallas import tpu as pltpu
from jax.experimental.pallas import tpu_sc as plsc

_NC, _NS = 2, 16
_NW = _NC * _NS
_CH = 400
_NB = 4
_G = 2


def _emb_body(x_hbm, w_hbm, out_hbm, idx_v, rows_v, gsem, ssem):
    wid = lax.axis_index("s") * _NC + lax.axis_index("c")
    k = idx_v.shape[0]
    d = w_hbm.shape[1]
    base = wid * k * _CH
    pltpu.sync_copy(x_hbm.at[pl.ds(wid * k, k)], idx_v)

    def fire_gather(j, slot):
        pltpu.async_copy(w_hbm.at[idx_v.at[j]], rows_v.at[slot], gsem)

    def fire_store(j, slot):
        pltpu.async_copy(rows_v.at[slot], out_hbm.at[pl.ds(base + j * _CH, _CH)], ssem)

    def drain_store():
        pltpu.make_async_copy(
            rows_v.at[0], out_hbm.at[pl.ds(base, _CH)], ssem).wait()

    def drain_gather(slot):
        pltpu.make_async_copy(
            w_hbm.at[idx_v.at[0]], rows_v.at[slot], gsem).wait()

    for b in range(_G):
        fire_gather(b, b)

    def outer(g, carry):
        for b in range(_NB):
            j = g * _NB + b

            @pl.when(j >= 1)
            def _():
                drain_store()

            @pl.when(j + _G < k)
            def _():
                fire_gather(j + _G, (b + _G) % _NB)

            drain_gather(b)
            fire_store(j, b)
        return carry

    lax.fori_loop(0, k // _NB, outer, 0)
    drain_store()


def kernel(x, W):
    b, s = x.shape
    v, d = W.shape
    n = b * s
    k = n // (_NW * _CH)
    x2 = x.reshape(_NW * k, _CH)
    mesh = plsc.VectorSubcoreMesh(core_axis_name="c", subcore_axis_name="s")
    out = pl.kernel(
        _emb_body,
        out_type=jax.ShapeDtypeStruct((n, d), jnp.float32),
        mesh=mesh,
        scratch_types=[
            pltpu.VMEM((k, _CH), jnp.int32),
            pltpu.VMEM((_NB, _CH, d), jnp.float32),
            pltpu.SemaphoreType.DMA,
            pltpu.SemaphoreType.DMA,
        ],
        compiler_params=pltpu.CompilerParams(use_tc_tiling_on_sc=False),
    )(x2, W)
    return out.reshape(b, s, d)
